# baseline (speedup 1.0000x reference)
import functools

import jax
import jax.numpy as jnp
from jax import lax
from jax.experimental import pallas as pl
from jax.experimental.pallas import tpu as pltpu
from jax.experimental.pallas import tpu_sc as plsc

_VOCAB = 1000000
_DIM = 64
_LANES = 16
_NC = 2
_NS = 16
_NW = _NC * _NS
_GRP = 128
_NBUF = 4
_NACC = 8
_RU = 4


def _make_bias_kernel(n_groups_total):
    n_groups_w = n_groups_total // _NW
    mesh = plsc.VectorSubcoreMesh(core_axis_name="c", subcore_axis_name="s")

    @functools.partial(
        pl.kernel,
        mesh=mesh,
        out_type=jax.ShapeDtypeStruct((_NW, _LANES), jnp.float32),
        compiler_params=pltpu.CompilerParams(use_tc_tiling_on_sc=False),
        scratch_types=[
            pltpu.VMEM((n_groups_w, _GRP), jnp.int32),
            pltpu.VMEM((n_groups_w, _GRP), jnp.int32),
            pltpu.VMEM((_NBUF, _GRP), jnp.float32),
            pltpu.VMEM((_NBUF, _GRP), jnp.float32),
            pltpu.VMEM((_LANES,), jnp.float32),
        ] + [pltpu.SemaphoreType.DMA] * _NBUF,
    )
    def bias_kernel(tok_hbm, head_hbm, ub_hbm, vb_hbm,
                    out_hbm, idx_t, idx_h, ub, vb, acc_v, *sems):
        wid = lax.axis_index("s") * _NC + lax.axis_index("c")
        g_base = wid * n_groups_w
        pltpu.sync_copy(tok_hbm.at[pl.ds(g_base, n_groups_w)], idx_t)
        pltpu.sync_copy(head_hbm.at[pl.ds(g_base, n_groups_w)], idx_h)

        def issue(gi, b):
            pltpu.async_copy(ub_hbm.at[idx_t.at[gi]], ub.at[b], sems[b])
            pltpu.async_copy(vb_hbm.at[idx_h.at[gi]], vb.at[b], sems[b])

        def drain(b):
            pltpu.make_async_copy(ub_hbm.at[pl.ds(0, _GRP)],
                                  ub.at[b], sems[b]).wait()
            pltpu.make_async_copy(vb_hbm.at[pl.ds(0, _GRP)],
                                  vb.at[b], sems[b]).wait()

        def compute(b, accs):
            a = list(accs)
            for j in range(_GRP // _LANES):
                sl = pl.ds(j * _LANES, _LANES)
                a[j] = a[j] + ub[b, sl] + vb[b, sl]
            return tuple(a)

        for b in range(_NBUF - 1):
            issue(b, b)

        def quad_body(q, accs):
            for b in range(_NBUF):
                gi = q * _NBUF + b
                issue(jnp.minimum(gi + _NBUF - 1, n_groups_w - 1),
                      (b + _NBUF - 1) % _NBUF)
                drain(b)
                accs = compute(b, accs)
            return accs

        accs = lax.fori_loop(
            0, n_groups_w // _NBUF, quad_body,
            tuple(jnp.zeros((_LANES,), jnp.float32) for _ in range(_NACC)))
        for b in range(_NBUF - 1):
            drain(b)

        total = accs[0]
        for a in accs[1:]:
            total = total + a
        acc_v[...] = total
        pltpu.sync_copy(acc_v, out_hbm.at[wid])

    return bias_kernel


def _make_vgather_kernel(n_groups_total):
    n_groups_w = n_groups_total // _NW
    mesh = plsc.VectorSubcoreMesh(core_axis_name="c", subcore_axis_name="s")

    @functools.partial(
        pl.kernel,
        mesh=mesh,
        out_type=jax.ShapeDtypeStruct((n_groups_total * _GRP, _DIM),
                                      jnp.float32),
        compiler_params=pltpu.CompilerParams(use_tc_tiling_on_sc=False),
        scratch_types=[
            pltpu.VMEM((n_groups_w, _GRP), jnp.int32),
            pltpu.VMEM((_NBUF, _GRP, _DIM), jnp.float32),
        ] + [pltpu.SemaphoreType.DMA] * (2 * _NBUF),
    )
    def vgather_kernel(head_hbm, v_hbm, out_hbm, idx_h, v_rows, *sems):
        semg = sems[:_NBUF]
        semo = sems[_NBUF:]
        wid = lax.axis_index("s") * _NC + lax.axis_index("c")
        g_base = wid * n_groups_w
        pltpu.sync_copy(head_hbm.at[pl.ds(g_base, n_groups_w)], idx_h)

        def issue(gi, b):
            pltpu.async_copy(v_hbm.at[idx_h.at[gi]], v_rows.at[b], semg[b])

        def drain(b):
            pltpu.make_async_copy(v_hbm.at[pl.ds(0, _GRP)],
                                  v_rows.at[b], semg[b]).wait()

        def issue_out(gi, b):
            row0 = (g_base + gi) * _GRP
            pltpu.async_copy(v_rows.at[b], out_hbm.at[pl.ds(row0, _GRP)],
                             semo[b])

        def drain_out(b):
            pltpu.make_async_copy(v_hbm.at[pl.ds(0, _GRP)],
                                  v_rows.at[b], semo[b]).wait()

        for b in range(_NBUF - 1):
            issue(b, b)

        def quad_body(q, carry):
            for b in range(_NBUF):
                gi = q * _NBUF + b
                nb = (b + _NBUF - 1) % _NBUF
                if b > 0:
                    drain_out(nb)
                else:
                    @pl.when(q > 0)
                    def _():
                        drain_out(nb)
                issue(jnp.minimum(gi + _NBUF - 1, n_groups_w - 1), nb)
                drain(b)
                issue_out(gi, b)
            return carry

        lax.fori_loop(0, n_groups_w // _NBUF, quad_body, 0)
        for b in range(_NBUF - 1):
            drain(b)
        drain_out(_NBUF - 1)

    return vgather_kernel


def _make_sc_kernel(n_groups_total):
    n_groups_w = n_groups_total // _NW
    assert n_groups_w % _NBUF == 0
    mesh = plsc.VectorSubcoreMesh(core_axis_name="c", subcore_axis_name="s")

    @functools.partial(
        pl.kernel,
        mesh=mesh,
        out_type=jax.ShapeDtypeStruct((_NW, _LANES), jnp.float32),
        compiler_params=pltpu.CompilerParams(use_tc_tiling_on_sc=False),
        scratch_types=[
            pltpu.VMEM((n_groups_w, _GRP), jnp.int32),
            pltpu.VMEM((_NBUF, _GRP, _DIM), jnp.float32),
            pltpu.VMEM((_NBUF, _GRP, _DIM), jnp.float32),
            pltpu.VMEM((_LANES,), jnp.float32),
        ] + [pltpu.SemaphoreType.DMA] * _NBUF,
    )
    def sc_kernel(tok_hbm, vg_hbm, u_hbm,
                  out_hbm, idx_t, u_rows, v_rows, acc_v,
                  *sems):
        wid = lax.axis_index("s") * _NC + lax.axis_index("c")
        g_base = wid * n_groups_w
        pltpu.sync_copy(tok_hbm.at[pl.ds(g_base, n_groups_w)], idx_t)

        def issue(gi, b):
            pltpu.async_copy(u_hbm.at[idx_t.at[gi]], u_rows.at[b], sems[b])
            pltpu.async_copy(vg_hbm.at[pl.ds((g_base + gi) * _GRP, _GRP)],
                             v_rows.at[b], sems[b])

        def drain(b):
            pltpu.make_async_copy(u_hbm.at[pl.ds(0, _GRP)],
                                  u_rows.at[b], sems[b]).wait()
            pltpu.make_async_copy(vg_hbm.at[pl.ds(0, _GRP)],
                                  v_rows.at[b], sems[b]).wait()

        def compute(b, accs):
            def row_body(i, a):
                a = list(a)
                for r in range(_RU):
                    for s in range(_DIM // _LANES):
                        sl = pl.ds(s * _LANES, _LANES)
                        k = (r % 2) * (_DIM // _LANES) + s
                        a[k] = a[k] + (u_rows[b, i * _RU + r, sl] *
                                       v_rows[b, i * _RU + r, sl])
                return tuple(a)

            return lax.fori_loop(0, _GRP // _RU, row_body, accs)

        for b in range(_NBUF - 1):
            issue(b, b)

        def quad_body(q, accs):
            for b in range(_NBUF):
                gi = q * _NBUF + b
                issue(jnp.minimum(gi + _NBUF - 1, n_groups_w - 1),
                      (b + _NBUF - 1) % _NBUF)
                drain(b)
                accs = compute(b, accs)
            return accs

        accs = lax.fori_loop(
            0, n_groups_w // _NBUF, quad_body,
            tuple(jnp.zeros((_LANES,), jnp.float32) for _ in range(_NACC)))
        for b in range(_NBUF - 1):
            drain(b)

        total = accs[0]
        for a in accs[1:]:
            total = total + a
        acc_v[...] = total
        pltpu.sync_copy(acc_v, out_hbm.at[wid])

    return sc_kernel


def kernel(tokens_batch, heads_batch, U, Ubias, V, Vbias):
    b, l = tokens_batch.shape
    n = b * l
    n_groups_total = n // _GRP
    tok = tokens_batch.reshape(n_groups_total, _GRP).astype(jnp.int32)
    head = heads_batch.reshape(n_groups_total, _GRP).astype(jnp.int32)
    ub_flat = Ubias.reshape(-1)
    vb_flat = Vbias.reshape(-1)
    bias_partials = _make_bias_kernel(n_groups_total)(
        tok, head, ub_flat, vb_flat)
    v_gath = _make_vgather_kernel(n_groups_total)(head, V)
    partials = _make_sc_kernel(n_groups_total)(tok, v_gath, U)
    return jnp.sum(partials) + jnp.sum(bias_partials)


# speedup vs baseline: 1.1098x; 1.1098x over previous
import functools

import jax
import jax.numpy as jnp
from jax import lax
from jax.experimental import pallas as pl
from jax.experimental.pallas import tpu as pltpu
from jax.experimental.pallas import tpu_sc as plsc

_VOCAB = 1000000
_DIM = 64
_LANES = 16
_NC = 2
_NS = 16
_NW = _NC * _NS
_GRP = 128
_NBUF = 4
_NACC = 8
_RU = 4


def _make_bias_kernel(n_groups_total):
    n_groups_w = n_groups_total // _NW
    mesh = plsc.VectorSubcoreMesh(core_axis_name="c", subcore_axis_name="s")

    @functools.partial(
        pl.kernel,
        mesh=mesh,
        out_type=jax.ShapeDtypeStruct((_NW, _LANES), jnp.float32),
        compiler_params=pltpu.CompilerParams(use_tc_tiling_on_sc=False),
        scratch_types=[
            pltpu.VMEM((n_groups_w, _GRP), jnp.int32),
            pltpu.VMEM((n_groups_w, _GRP), jnp.int32),
            pltpu.VMEM((_NBUF, _GRP), jnp.float32),
            pltpu.VMEM((_NBUF, _GRP), jnp.float32),
            pltpu.VMEM((_LANES,), jnp.float32),
        ] + [pltpu.SemaphoreType.DMA] * _NBUF,
    )
    def bias_kernel(tok_hbm, head_hbm, ub_hbm, vb_hbm,
                    out_hbm, idx_t, idx_h, ub, vb, acc_v, *sems):
        wid = lax.axis_index("s") * _NC + lax.axis_index("c")
        g_base = wid * n_groups_w
        pltpu.sync_copy(tok_hbm.at[pl.ds(g_base, n_groups_w)], idx_t)
        pltpu.sync_copy(head_hbm.at[pl.ds(g_base, n_groups_w)], idx_h)

        def issue(gi, b):
            pltpu.async_copy(ub_hbm.at[idx_t.at[gi]], ub.at[b], sems[b])
            pltpu.async_copy(vb_hbm.at[idx_h.at[gi]], vb.at[b], sems[b])

        def drain(b):
            pltpu.make_async_copy(ub_hbm.at[pl.ds(0, _GRP)],
                                  ub.at[b], sems[b]).wait()
            pltpu.make_async_copy(vb_hbm.at[pl.ds(0, _GRP)],
                                  vb.at[b], sems[b]).wait()

        def compute(b, accs):
            a = list(accs)
            for j in range(_GRP // _LANES):
                sl = pl.ds(j * _LANES, _LANES)
                a[j] = a[j] + ub[b, sl] + vb[b, sl]
            return tuple(a)

        for b in range(_NBUF - 1):
            issue(b, b)

        def quad_body(q, accs):
            for b in range(_NBUF):
                gi = q * _NBUF + b
                issue(jnp.minimum(gi + _NBUF - 1, n_groups_w - 1),
                      (b + _NBUF - 1) % _NBUF)
                drain(b)
                accs = compute(b, accs)
            return accs

        accs = lax.fori_loop(
            0, n_groups_w // _NBUF, quad_body,
            tuple(jnp.zeros((_LANES,), jnp.float32) for _ in range(_NACC)))
        for b in range(_NBUF - 1):
            drain(b)

        total = accs[0]
        for a in accs[1:]:
            total = total + a
        acc_v[...] = total
        pltpu.sync_copy(acc_v, out_hbm.at[wid])

    return bias_kernel


def _make_sc_kernel(n_groups_total):
    n_groups_w = n_groups_total // _NW
    assert n_groups_w % _NBUF == 0
    mesh = plsc.VectorSubcoreMesh(core_axis_name="c", subcore_axis_name="s")

    @functools.partial(
        pl.kernel,
        mesh=mesh,
        out_type=jax.ShapeDtypeStruct((_NW, _LANES), jnp.float32),
        compiler_params=pltpu.CompilerParams(use_tc_tiling_on_sc=False),
        scratch_types=[
            pltpu.VMEM((n_groups_w, _GRP), jnp.int32),
            pltpu.VMEM((n_groups_w, _GRP), jnp.int32),
            pltpu.VMEM((_NBUF, _GRP, _DIM), jnp.float32),
            pltpu.VMEM((_NBUF, _GRP, _DIM), jnp.float32),
            pltpu.VMEM((_LANES,), jnp.float32),
        ] + [pltpu.SemaphoreType.DMA] * _NBUF,
    )
    def sc_kernel(tok_hbm, head_hbm, u_hbm, v_hbm,
                  out_hbm, idx_t, idx_h, u_rows, v_rows, acc_v,
                  *sems):
        wid = lax.axis_index("s") * _NC + lax.axis_index("c")
        g_base = wid * n_groups_w
        pltpu.sync_copy(tok_hbm.at[pl.ds(g_base, n_groups_w)], idx_t)
        pltpu.sync_copy(head_hbm.at[pl.ds(g_base, n_groups_w)], idx_h)

        def issue(gi, b):
            pltpu.async_copy(u_hbm.at[idx_t.at[gi]], u_rows.at[b], sems[b])
            pltpu.async_copy(v_hbm.at[idx_h.at[gi]], v_rows.at[b], sems[b])

        def drain(b):
            pltpu.make_async_copy(u_hbm.at[pl.ds(0, _GRP)],
                                  u_rows.at[b], sems[b]).wait()
            pltpu.make_async_copy(v_hbm.at[pl.ds(0, _GRP)],
                                  v_rows.at[b], sems[b]).wait()

        def compute(b, accs):
            def row_body(i, a):
                a = list(a)
                for r in range(_RU):
                    for s in range(_DIM // _LANES):
                        sl = pl.ds(s * _LANES, _LANES)
                        k = (r % 2) * (_DIM // _LANES) + s
                        a[k] = a[k] + (u_rows[b, i * _RU + r, sl] *
                                       v_rows[b, i * _RU + r, sl])
                return tuple(a)

            return lax.fori_loop(0, _GRP // _RU, row_body, accs)

        for b in range(_NBUF - 1):
            issue(b, b)

        def quad_body(q, accs):
            for b in range(_NBUF):
                gi = q * _NBUF + b
                issue(jnp.minimum(gi + _NBUF - 1, n_groups_w - 1),
                      (b + _NBUF - 1) % _NBUF)
                drain(b)
                accs = compute(b, accs)
            return accs

        accs = lax.fori_loop(
            0, n_groups_w // _NBUF, quad_body,
            tuple(jnp.zeros((_LANES,), jnp.float32) for _ in range(_NACC)))
        for b in range(_NBUF - 1):
            drain(b)

        total = accs[0]
        for a in accs[1:]:
            total = total + a
        acc_v[...] = total
        pltpu.sync_copy(acc_v, out_hbm.at[wid])

    return sc_kernel


def kernel(tokens_batch, heads_batch, U, Ubias, V, Vbias):
    b, l = tokens_batch.shape
    n = b * l
    n_groups_total = n // _GRP
    tok = tokens_batch.reshape(n_groups_total, _GRP).astype(jnp.int32)
    head = heads_batch.reshape(n_groups_total, _GRP).astype(jnp.int32)
    ub_flat = Ubias.reshape(-1)
    vb_flat = Vbias.reshape(-1)
    bias_partials = _make_bias_kernel(n_groups_total)(
        tok, head, ub_flat, vb_flat)
    partials = _make_sc_kernel(n_groups_total)(tok, head, U, V)
    return jnp.sum(partials) + jnp.sum(bias_partials)
